# baseline (speedup 1.0000x reference)
import functools

import jax
import jax.numpy as jnp
from jax import lax
from jax.experimental import pallas as pl
from jax.experimental.pallas import tpu as pltpu
from jax.experimental.pallas import tpu_sc as plsc

_N = 10000
_E = 320000
_F = 128
_ES = 20
_CUT = 5.0

_NP = 10240
_NT = 16
_B = 32
_EPT = _E // _NT
_NBATCH = _EPT // _B
_RPT = _NP // _NT
_CHK = 25 * _B


def _node_body(ns_ref, nv_ref, w1_ref, b1_ref, w2_ref, b2_ref,
               so_gs_ref, so_ge_ref, so_ms_ref, nvx_ref, nvy_ref, nvz_ref):
    h = jnp.dot(ns_ref[...], w1_ref[...], preferred_element_type=jnp.float32)
    h = h + b1_ref[...]
    h = h * jax.nn.sigmoid(h)
    so = jnp.dot(h, w2_ref[...], preferred_element_type=jnp.float32)
    so = so + b2_ref[...]
    so_gs_ref[...] = so[:, 0:_F]
    so_ge_ref[...] = so[:, _F:2 * _F]
    so_ms_ref[...] = so[:, 2 * _F:3 * _F]
    nv = nv_ref[...]
    nvx_ref[...] = nv[:, 0, :]
    nvy_ref[...] = nv[:, 1, :]
    nvz_ref[...] = nv[:, 2, :]


def _node_precompute(node_scalar, node_vector, w1, b1r, w2, b2r):
    nb = 1024
    grid = (_NP // nb,)
    f32 = jnp.float32
    return pl.pallas_call(
        _node_body,
        grid=grid,
        in_specs=[
            pl.BlockSpec((nb, _F), lambda i: (i, 0)),
            pl.BlockSpec((nb, 3, _F), lambda i: (i, 0, 0)),
            pl.BlockSpec((_F, _F), lambda i: (0, 0)),
            pl.BlockSpec((1, _F), lambda i: (0, 0)),
            pl.BlockSpec((_F, 3 * _F), lambda i: (0, 0)),
            pl.BlockSpec((1, 3 * _F), lambda i: (0, 0)),
        ],
        out_specs=[pl.BlockSpec((nb, _F), lambda i: (i, 0))] * 6,
        out_shape=[jax.ShapeDtypeStruct((_NP, _F), f32)] * 6,
    )(node_scalar, node_vector, w1, b1r, w2, b2r)


def _edge_body(d_ref, diff_ref, wf_ref, bf_ref,
               fgs_ref, fms_ref, fdx_ref, fdy_ref, fdz_ref):
    d = d_ref[...]
    n = (lax.broadcasted_iota(jnp.int32, (1, _ES), 1) + 1).astype(jnp.float32)

    def _sin_reduced(r):
        r2 = r * r
        return r * (1.0 + r2 * (-1.0 / 6.0 + r2 * (1.0 / 120.0 + r2 * (
            -1.0 / 5040.0 + r2 * (1.0 / 362880.0)))))

    x = d * (n * (jnp.pi / _CUT))
    k = jnp.floor(x * (1.0 / jnp.pi) + 0.5)
    r = x - k * jnp.pi
    odd = (k.astype(jnp.int32) & 1) != 0
    sinx = jnp.where(odd, -1.0, 1.0) * _sin_reduced(r)
    rbf = sinx / d
    w = jnp.dot(rbf, wf_ref[...], preferred_element_type=jnp.float32)
    w = w + bf_ref[...]
    cosd = _sin_reduced(jnp.pi / 2.0 - d * (jnp.pi / _CUT))
    fcut = jnp.where(d < _CUT, 0.5 * (cosd + 1.0), 0.0)
    w = w * fcut
    inv_d = 1.0 / d
    diff = diff_ref[...]
    ge = w[:, _F:2 * _F]
    fgs_ref[...] = w[:, 0:_F]
    fms_ref[...] = w[:, 2 * _F:3 * _F]
    fdx_ref[...] = ge * (diff[:, 0:1] * inv_d)
    fdy_ref[...] = ge * (diff[:, 1:2] * inv_d)
    fdz_ref[...] = ge * (diff[:, 2:3] * inv_d)


def _edge_filter(dist2, diff, wf, bfr):
    eb = 2000
    grid = (_E // eb,)
    f32 = jnp.float32
    return pl.pallas_call(
        _edge_body,
        grid=grid,
        in_specs=[
            pl.BlockSpec((eb, 1), lambda i: (i, 0)),
            pl.BlockSpec((eb, 3), lambda i: (i, 0)),
            pl.BlockSpec((_ES, 3 * _F), lambda i: (0, 0)),
            pl.BlockSpec((1, 3 * _F), lambda i: (0, 0)),
        ],
        out_specs=[pl.BlockSpec((eb, _F), lambda i: (i, 0))] * 5,
        out_shape=[jax.ShapeDtypeStruct((_E, _F), f32)] * 5,
    )(dist2, diff, wf, bfr)


def _sc_body(so_gs, so_ge, so_ms, nvx, nvy, nvz, ns_in,
             fgs, fms, fdx, fdy, fdz, src, dst,
             out_s, out_vx, out_vy, out_vz,
             src_c, dst_v0, dst_v1,
             ga0, ga1, ga2, la0, la1,
             gb0, gb1, gb2, lb0, lb1,
             acc, sem0, sem1, dsem0, dsem1, ssem):
    c = lax.axis_index("c")
    s = lax.axis_index("s")
    row0 = s * _RPT
    ebase = s * _EPT
    dst_v = (dst_v0, dst_v1)
    dsem = (dsem0, dsem1)
    g0 = (ga0, gb0)
    g1 = (ga1, gb1)
    g2 = (ga2, gb2)
    l0 = (la0, lb0)
    l1 = (la1, lb1)
    sem = (sem0, sem1)

    def compute_scalar(cur):
        @plsc.parallel_loop(0, _B, unroll=4)
        def _(b):
            for j in range(_F // 16):
                sl = pl.ds(j * 16, 16)
                g0[cur][b, sl] = g0[cur][b, sl] * l0[cur][b, sl]

    def compute_vec(cur):
        @plsc.parallel_loop(0, _B, unroll=4)
        def _(b):
            for j in range(_F // 16):
                sl = pl.ds(j * 16, 16)
                g2[cur][b, sl] = g2[cur][b, sl] * (g0[cur][b, sl] * l0[cur][b, sl]) \
                    + g1[cur][b, sl] * l1[cur][b, sl]

    def sweep(init_hbm, out_hbm, gathers, linears, compute_fn, scat):
        pltpu.sync_copy(init_hbm.at[pl.ds(row0, _RPT)],
                        acc.at[pl.ds(row0, _RPT)])
        plsc.subcore_barrier()

        pltpu.sync_copy(src.at[pl.ds(ebase, _CHK)], src_c)
        pltpu.async_copy(dst.at[pl.ds(ebase, _B)], dst_v0, dsem0)
        pltpu.async_copy(dst.at[pl.ds(ebase + _B, _B)], dst_v1, dsem1)

        def descs(sidx, e1, slot):
            d = [pltpu.make_async_copy(tbl.at[sidx], bufs[slot], sem[slot])
                 for (tbl, bufs) in gathers]
            d += [pltpu.make_async_copy(arr.at[pl.ds(e1, _B)], bufs[slot],
                                        sem[slot])
                  for (arr, bufs) in linears]
            return d

        for d in descs(src_c.at[pl.ds(0, _B)], ebase, 0):
            d.start()
            d.wait()

        def body(i, cur):
            nxt = 1 - cur
            e0 = ebase + i * _B
            e1 = e0 + _B
            r1 = (i + 1) % 25

            @pl.when(jnp.logical_and(r1 == 0, i + 1 < _NBATCH))
            def _():
                pltpu.sync_copy(
                    src.at[pl.ds(ebase + ((i + 1) // 25) * _CHK, _CHK)],
                    src_c)

            nds = descs(src_c.at[pl.ds(r1 * _B, _B)], e1, nxt)

            @pl.when(i + 1 < _NBATCH)
            def _():
                for d in nds:
                    d.start()

            compute_fn(cur)
            pltpu.make_async_copy(dst.at[pl.ds(e0, _B)], dst_v[cur],
                                  dsem[cur]).wait()
            ssd = pltpu.async_copy(scat[cur], acc.at[dst_v[cur]], ssem,
                                   add=True)

            @pl.when(i + 1 < _NBATCH)
            def _():
                for d in nds:
                    d.wait()

            ssd.wait()

            @pl.when(i + 2 < _NBATCH)
            def _():
                pltpu.async_copy(dst.at[pl.ds(e0 + 2 * _B, _B)], dst_v[cur],
                                 dsem[cur])

        def pair(p, carry):
            body(2 * p, 0)
            body(2 * p + 1, 1)
            return carry

        lax.fori_loop(0, _NBATCH // 2, pair, 0)
        body(_NBATCH - 1, 0)
        plsc.subcore_barrier()
        pltpu.sync_copy(acc.at[pl.ds(row0, _RPT)],
                        out_hbm.at[pl.ds(row0, _RPT)])

    def sweep_scalar(init_hbm, out_hbm):
        sweep(init_hbm, out_hbm, [(so_ms, g0)], [(fms, l0)],
              compute_scalar, g0)

    def sweep_vec(nvk, fdk, out_hbm):
        sweep(nvk, out_hbm, [(so_gs, g0), (so_ge, g1), (nvk, g2)],
              [(fgs, l0), (fdk, l1)], compute_vec, g2)

    @pl.when(c == 0)
    def _():
        sweep_scalar(ns_in, out_s)
        sweep_vec(nvy, fdy, out_vy)

    @pl.when(c == 1)
    def _():
        sweep_vec(nvx, fdx, out_vx)
        sweep_vec(nvz, fdz, out_vz)


def _sc_scatter(so_gs, so_ge, so_ms, nvx, nvy, nvz, ns,
                fgs, fms, fdx, fdy, fdz, src, dst):
    f32 = jnp.float32
    mesh = plsc.VectorSubcoreMesh(core_axis_name="c", subcore_axis_name="s",
                                  num_cores=2, num_subcores=_NT)
    idx_t = pltpu.VMEM((_B,), jnp.int32)
    buf_t = pltpu.VMEM((_B, _F), f32)
    fn = pl.kernel(
        _sc_body,
        out_type=[jax.ShapeDtypeStruct((_NP, _F), f32)] * 4,
        mesh=mesh,
        scratch_types=[pltpu.VMEM((_CHK,), jnp.int32), idx_t, idx_t]
        + [buf_t] * 10 + [
            pltpu.VMEM_SHARED((_NP, _F), f32),
            pltpu.SemaphoreType.DMA,
            pltpu.SemaphoreType.DMA,
            pltpu.SemaphoreType.DMA,
            pltpu.SemaphoreType.DMA,
            pltpu.SemaphoreType.DMA,
        ],
    )
    return fn(so_gs, so_ge, so_ms, nvx, nvy, nvz, ns,
              fgs, fms, fdx, fdy, fdz, src, dst)


def kernel(node_scalar, node_vector, edge, edge_diff, edge_dist,
           W_filter, b_filter, W1, b1, W2, b2):
    src = edge[:, 1]
    dst = edge[:, 0]
    pad = _NP - _N
    ns_p = jnp.pad(node_scalar, ((0, pad), (0, 0)))
    nv_p = jnp.pad(node_vector, ((0, pad), (0, 0), (0, 0)))
    so_gs, so_ge, so_ms, nvx, nvy, nvz = _node_precompute(
        ns_p, nv_p, W1, b1.reshape(1, _F), W2, b2.reshape(1, 3 * _F))
    fgs, fms, fdx, fdy, fdz = _edge_filter(
        edge_dist.reshape(_E, 1), edge_diff, W_filter,
        b_filter.reshape(1, 3 * _F))
    out_s, out_vx, out_vy, out_vz = _sc_scatter(
        so_gs, so_ge, so_ms, nvx, nvy, nvz, ns_p,
        fgs, fms, fdx, fdy, fdz, src, dst)
    new_vec = jnp.stack([out_vx[:_N], out_vy[:_N], out_vz[:_N]], axis=1)
    return (out_s[:_N], new_vec)


# speedup vs baseline: 1.0000x; 1.0000x over previous
import functools

import jax
import jax.numpy as jnp
from jax import lax
from jax.experimental import pallas as pl
from jax.experimental.pallas import tpu as pltpu
from jax.experimental.pallas import tpu_sc as plsc

_N = 10000
_E = 320000
_F = 128
_ES = 20
_CUT = 5.0

_NP = 10240
_NT = 16
_B = 32
_EPT = _E // _NT
_NBATCH = _EPT // _B
_RPT = _NP // _NT
_CHK = 25 * _B


def _node_body(ns_ref, nv_ref, w1_ref, b1_ref, w2_ref, b2_ref,
               so_gsge_ref, so_ms_ref, nvx_ref, nvy_ref, nvz_ref):
    h = jnp.dot(ns_ref[...], w1_ref[...], preferred_element_type=jnp.float32)
    h = h + b1_ref[...]
    h = h * jax.nn.sigmoid(h)
    so = jnp.dot(h, w2_ref[...], preferred_element_type=jnp.float32)
    so = so + b2_ref[...]
    so_gsge_ref[...] = so[:, 0:2 * _F]
    so_ms_ref[...] = so[:, 2 * _F:3 * _F]
    nv = nv_ref[...]
    nvx_ref[...] = nv[:, 0, :]
    nvy_ref[...] = nv[:, 1, :]
    nvz_ref[...] = nv[:, 2, :]


def _node_precompute(node_scalar, node_vector, w1, b1r, w2, b2r):
    nb = 1024
    grid = (_NP // nb,)
    f32 = jnp.float32
    return pl.pallas_call(
        _node_body,
        grid=grid,
        in_specs=[
            pl.BlockSpec((nb, _F), lambda i: (i, 0)),
            pl.BlockSpec((nb, 3, _F), lambda i: (i, 0, 0)),
            pl.BlockSpec((_F, _F), lambda i: (0, 0)),
            pl.BlockSpec((1, _F), lambda i: (0, 0)),
            pl.BlockSpec((_F, 3 * _F), lambda i: (0, 0)),
            pl.BlockSpec((1, 3 * _F), lambda i: (0, 0)),
        ],
        out_specs=[pl.BlockSpec((nb, 2 * _F), lambda i: (i, 0))]
        + [pl.BlockSpec((nb, _F), lambda i: (i, 0))] * 4,
        out_shape=[jax.ShapeDtypeStruct((_NP, 2 * _F), f32)]
        + [jax.ShapeDtypeStruct((_NP, _F), f32)] * 4,
    )(node_scalar, node_vector, w1, b1r, w2, b2r)


def _edge_body(d_ref, diff_ref, wf_ref, bf_ref,
               fgs_ref, fms_ref, fdx_ref, fdy_ref, fdz_ref):
    d = d_ref[...]
    n = (lax.broadcasted_iota(jnp.int32, (1, _ES), 1) + 1).astype(jnp.float32)

    def _sin_reduced(r):
        r2 = r * r
        return r * (1.0 + r2 * (-1.0 / 6.0 + r2 * (1.0 / 120.0 + r2 * (
            -1.0 / 5040.0 + r2 * (1.0 / 362880.0)))))

    x = d * (n * (jnp.pi / _CUT))
    k = jnp.floor(x * (1.0 / jnp.pi) + 0.5)
    r = x - k * jnp.pi
    odd = (k.astype(jnp.int32) & 1) != 0
    sinx = jnp.where(odd, -1.0, 1.0) * _sin_reduced(r)
    rbf = sinx / d
    w = jnp.dot(rbf, wf_ref[...], preferred_element_type=jnp.float32)
    w = w + bf_ref[...]
    cosd = _sin_reduced(jnp.pi / 2.0 - d * (jnp.pi / _CUT))
    fcut = jnp.where(d < _CUT, 0.5 * (cosd + 1.0), 0.0)
    w = w * fcut
    inv_d = 1.0 / d
    diff = diff_ref[...]
    ge = w[:, _F:2 * _F]
    fgs_ref[...] = w[:, 0:_F]
    fms_ref[...] = w[:, 2 * _F:3 * _F]
    fdx_ref[...] = ge * (diff[:, 0:1] * inv_d)
    fdy_ref[...] = ge * (diff[:, 1:2] * inv_d)
    fdz_ref[...] = ge * (diff[:, 2:3] * inv_d)


def _edge_filter(dist2, diff, wf, bfr):
    eb = 2000
    grid = (_E // eb,)
    f32 = jnp.float32
    return pl.pallas_call(
        _edge_body,
        grid=grid,
        in_specs=[
            pl.BlockSpec((eb, 1), lambda i: (i, 0)),
            pl.BlockSpec((eb, 3), lambda i: (i, 0)),
            pl.BlockSpec((_ES, 3 * _F), lambda i: (0, 0)),
            pl.BlockSpec((1, 3 * _F), lambda i: (0, 0)),
        ],
        out_specs=[pl.BlockSpec((eb, _F), lambda i: (i, 0))] * 5,
        out_shape=[jax.ShapeDtypeStruct((_E, _F), f32)] * 5,
    )(dist2, diff, wf, bfr)


def _sc_body(so_gsge, so_ms, nvx, nvy, nvz, ns_in,
             fgs, fms, fdx, fdy, fdz, src, dst,
             out_s, out_vx, out_vy, out_vz,
             src_c, dst_v0, dst_v1,
             ga0, ga2, la0, la1,
             gb0, gb2, lb0, lb1,
             acc, sem0, sem1, dsem0, dsem1, ssem):
    c = lax.axis_index("c")
    s = lax.axis_index("s")
    row0 = s * _RPT
    ebase = s * _EPT
    dst_v = (dst_v0, dst_v1)
    dsem = (dsem0, dsem1)
    gg = (ga0, gb0)
    g2 = (ga2, gb2)
    l0 = (la0, lb0)
    l1 = (la1, lb1)
    sem = (sem0, sem1)

    def compute_scalar(cur):
        @plsc.parallel_loop(0, _B, unroll=4)
        def _(b):
            for j in range(_F // 16):
                sl = pl.ds(j * 16, 16)
                g2[cur][b, sl] = g2[cur][b, sl] * l0[cur][b, sl]

    def compute_vec(cur):
        @plsc.parallel_loop(0, _B, unroll=4)
        def _(b):
            for j in range(_F // 16):
                sl = pl.ds(j * 16, 16)
                sl2 = pl.ds(_F + j * 16, 16)
                g2[cur][b, sl] = g2[cur][b, sl] * (gg[cur][b, sl] * l0[cur][b, sl]) \
                    + gg[cur][b, sl2] * l1[cur][b, sl]

    def sweep(init_hbm, out_hbm, gathers, linears, compute_fn, scat):
        pltpu.sync_copy(init_hbm.at[pl.ds(row0, _RPT)],
                        acc.at[pl.ds(row0, _RPT)])
        plsc.subcore_barrier()

        pltpu.sync_copy(src.at[pl.ds(ebase, _CHK)], src_c)
        pltpu.async_copy(dst.at[pl.ds(ebase, _B)], dst_v0, dsem0)
        pltpu.async_copy(dst.at[pl.ds(ebase + _B, _B)], dst_v1, dsem1)

        def descs(sidx, e1, slot):
            d = [pltpu.make_async_copy(tbl.at[sidx], bufs[slot], sem[slot])
                 for (tbl, bufs) in gathers]
            d += [pltpu.make_async_copy(arr.at[pl.ds(e1, _B)], bufs[slot],
                                        sem[slot])
                  for (arr, bufs) in linears]
            return d

        for d in descs(src_c.at[pl.ds(0, _B)], ebase, 0):
            d.start()
            d.wait()

        def body(i, cur):
            nxt = 1 - cur
            e0 = ebase + i * _B
            e1 = e0 + _B
            r1 = (i + 1) % 25

            @pl.when(jnp.logical_and(r1 == 0, i + 1 < _NBATCH))
            def _():
                pltpu.sync_copy(
                    src.at[pl.ds(ebase + ((i + 1) // 25) * _CHK, _CHK)],
                    src_c)

            nds = descs(src_c.at[pl.ds(r1 * _B, _B)], e1, nxt)

            @pl.when(i + 1 < _NBATCH)
            def _():
                for d in nds:
                    d.start()

            compute_fn(cur)
            pltpu.make_async_copy(dst.at[pl.ds(e0, _B)], dst_v[cur],
                                  dsem[cur]).wait()
            ssd = pltpu.async_copy(scat[cur], acc.at[dst_v[cur]], ssem,
                                   add=True)

            @pl.when(i + 1 < _NBATCH)
            def _():
                for d in nds:
                    d.wait()

            ssd.wait()

            @pl.when(i + 2 < _NBATCH)
            def _():
                pltpu.async_copy(dst.at[pl.ds(e0 + 2 * _B, _B)], dst_v[cur],
                                 dsem[cur])

        def pair(p, carry):
            body(2 * p, 0)
            body(2 * p + 1, 1)
            return carry

        lax.fori_loop(0, _NBATCH // 2, pair, 0)
        body(_NBATCH - 1, 0)
        plsc.subcore_barrier()
        pltpu.sync_copy(acc.at[pl.ds(row0, _RPT)],
                        out_hbm.at[pl.ds(row0, _RPT)])

    def sweep_scalar(init_hbm, out_hbm):
        sweep(init_hbm, out_hbm, [(so_ms, g2)], [(fms, l0)],
              compute_scalar, g2)

    def sweep_vec(nvk, fdk, out_hbm):
        sweep(nvk, out_hbm, [(so_gsge, gg), (nvk, g2)],
              [(fgs, l0), (fdk, l1)], compute_vec, g2)

    @pl.when(c == 0)
    def _():
        sweep_scalar(ns_in, out_s)
        sweep_vec(nvy, fdy, out_vy)

    @pl.when(c == 1)
    def _():
        sweep_vec(nvx, fdx, out_vx)
        sweep_vec(nvz, fdz, out_vz)


def _sc_scatter(so_gsge, so_ms, nvx, nvy, nvz, ns,
                fgs, fms, fdx, fdy, fdz, src, dst):
    f32 = jnp.float32
    mesh = plsc.VectorSubcoreMesh(core_axis_name="c", subcore_axis_name="s",
                                  num_cores=2, num_subcores=_NT)
    idx_t = pltpu.VMEM((_B,), jnp.int32)
    buf_t = pltpu.VMEM((_B, _F), f32)
    gg_t = pltpu.VMEM((_B, 2 * _F), f32)
    fn = pl.kernel(
        _sc_body,
        out_type=[jax.ShapeDtypeStruct((_NP, _F), f32)] * 4,
        mesh=mesh,
        scratch_types=[pltpu.VMEM((_CHK,), jnp.int32), idx_t, idx_t]
        + [gg_t, buf_t, buf_t, buf_t] * 2 + [
            pltpu.VMEM_SHARED((_NP, _F), f32),
            pltpu.SemaphoreType.DMA,
            pltpu.SemaphoreType.DMA,
            pltpu.SemaphoreType.DMA,
            pltpu.SemaphoreType.DMA,
            pltpu.SemaphoreType.DMA,
        ],
    )
    return fn(so_gsge, so_ms, nvx, nvy, nvz, ns,
              fgs, fms, fdx, fdy, fdz, src, dst)


def kernel(node_scalar, node_vector, edge, edge_diff, edge_dist,
           W_filter, b_filter, W1, b1, W2, b2):
    src = edge[:, 1]
    dst = edge[:, 0]
    pad = _NP - _N
    ns_p = jnp.pad(node_scalar, ((0, pad), (0, 0)))
    nv_p = jnp.pad(node_vector, ((0, pad), (0, 0), (0, 0)))
    so_gsge, so_ms, nvx, nvy, nvz = _node_precompute(
        ns_p, nv_p, W1, b1.reshape(1, _F), W2, b2.reshape(1, 3 * _F))
    fgs, fms, fdx, fdy, fdz = _edge_filter(
        edge_dist.reshape(_E, 1), edge_diff, W_filter,
        b_filter.reshape(1, 3 * _F))
    out_s, out_vx, out_vy, out_vz = _sc_scatter(
        so_gsge, so_ms, nvx, nvy, nvz, ns_p,
        fgs, fms, fdx, fdy, fdz, src, dst)
    new_vec = jnp.stack([out_vx[:_N], out_vy[:_N], out_vz[:_N]], axis=1)
    return (out_s[:_N], new_vec)
